# baseline (speedup 1.0000x reference)
import functools

import jax
import jax.numpy as jnp
from jax import lax
from jax.experimental import pallas as pl
from jax.experimental.pallas import tpu as pltpu
from jax.experimental.pallas import tpu_sc as plsc

N_SPINS = 20
BATCH = 16384
NUM_WORKERS = 32
B_PER_W = BATCH // NUM_WORKERS
ROWS = 4


def _sc_body(sig_hbm, amps_hbm, out_hbm, sig_v, idx_v, out_v,
             s0, s1, s2, s3, gsem):
    nc = 2
    wid = lax.axis_index("s") * nc + lax.axis_index("c")
    base = wid * B_PER_W
    sems = [s0, s1, s2, s3]

    loads = [
        pltpu.async_copy(sig_hbm.at[:, pl.ds(base + r * 128, 128)],
                         sig_v.at[:, pl.ds(r * 128, 128)], sems[r])
        for r in range(ROWS)
    ]

    zeros = jnp.zeros((16,), jnp.int32)
    gathers = []
    for r in range(ROWS):
        loads[r].wait()

        def group_body(j, _, r=r):
            c0 = r * 128 + j * 16
            acc = zeros
            for i in range(N_SPINS):
                v = sig_v[i, pl.ds(c0, 16)]
                acc = acc | jnp.where(v > 0.0,
                                      jnp.full((16,), 1 << i, jnp.int32),
                                      zeros)
            idx_v[pl.ds(c0, 16)] = acc
            return 0

        lax.fori_loop(0, 128 // 16, group_body, 0, unroll=False)
        gathers.append(
            pltpu.async_copy(amps_hbm.at[idx_v.at[pl.ds(r * 128, 128)]],
                             out_v.at[pl.ds(r * 128, 128)], gsem))
    for cp in gathers:
        cp.wait()

    pltpu.sync_copy(out_v, out_hbm.at[pl.ds(base, B_PER_W)])


@jax.jit
def kernel(sigma, amps):
    sig_t = sigma.T
    mesh = plsc.VectorSubcoreMesh(core_axis_name="c", subcore_axis_name="s")
    k = functools.partial(
        pl.kernel,
        mesh=mesh,
        out_type=jax.ShapeDtypeStruct((BATCH,), jnp.float32),
        scratch_types=[
            pltpu.VMEM((N_SPINS, B_PER_W), jnp.float32),
            pltpu.VMEM((B_PER_W,), jnp.int32),
            pltpu.VMEM((B_PER_W,), jnp.float32),
            pltpu.SemaphoreType.DMA,
            pltpu.SemaphoreType.DMA,
            pltpu.SemaphoreType.DMA,
            pltpu.SemaphoreType.DMA,
            pltpu.SemaphoreType.DMA,
        ],
        compiler_params=pltpu.CompilerParams(needs_layout_passes=False),
    )(_sc_body)
    return k(sig_t, amps)
